# baseline (speedup 1.0000x reference)
import functools

import jax
import jax.numpy as jnp
from jax import lax
from jax.experimental import pallas as pl
from jax.experimental.pallas import tpu as pltpu
from jax.experimental.pallas import tpu_sc as plsc

NUM_CLASSES = 19
KP = 24
EPS = 1e-05
COUNT = 6
B, C, H, W = 4, 96, 224, 224
HW = H * W
T = 7168
NB = HW // T

NW = 32
GPB = NW // B
CPW = C // GPB
PCH = 1792
NCH = HW // PCH
VPC = PCH // 16
KP2 = 32
REG = 16 * KP2
NREG = CPW * 2 + 1
ACC_N = NREG * REG
RES_N = NREG * KP2


def _sc_stats_body(x_hbm, lab_hbm, out_hbm, xbuf, lbuf, acc, res, sems):
    wid = lax.axis_index("s") * 2 + lax.axis_index("c")
    b = wid // GPB
    g = wid % GPB
    row0 = b * C + g * CPW
    lane32 = lax.iota(jnp.int32, 16) * KP2
    ones = jnp.ones((16,), jnp.float32)
    zeros = jnp.zeros((16,), jnp.float32)

    def zero_body(i, _):
        acc[pl.ds(i * 16, 16)] = zeros
        return 0
    lax.fori_loop(0, ACC_N // 16, zero_body, 0)

    def start_chunk(t, slot):
        for c in range(CPW):
            pltpu.make_async_copy(
                x_hbm.at[pl.ds((row0 + c) * HW + t * PCH, PCH)],
                xbuf.at[slot, c], sems.at[slot]).start()
        pltpu.make_async_copy(
            lab_hbm.at[pl.ds(b * HW + t * PCH, PCH)],
            lbuf.at[slot], sems.at[slot]).start()

    start_chunk(0, 0)

    def wait_chunk(slot):
        for c in range(CPW):
            pltpu.make_async_copy(
                x_hbm.at[pl.ds(0, PCH)],
                xbuf.at[slot, c], sems.at[slot]).wait()
        pltpu.make_async_copy(
            lab_hbm.at[pl.ds(0, PCH)],
            lbuf.at[slot], sems.at[slot]).wait()

    def chunk_body(t, _):
        slot = lax.rem(t, 2)

        @pl.when(t + 1 < NCH)
        def _():
            start_chunk(t + 1, 1 - slot)

        wait_chunk(slot)

        def vreg_body(v, _):
            pv = v * 16
            labv = lbuf[slot, pl.ds(pv, 16)]
            idx0 = lax.iota(jnp.int32, 16) * KP2 + labv
            plsc.addupdate_scatter(
                acc, [idx0 + ((NREG - 1) * REG)], jnp.ones((16,), jnp.float32))
            for c in range(CPW):
                xv = xbuf[slot, c, pl.ds(pv, 16)]
                idx_c = idx0 + (2 * c * REG)
                plsc.addupdate_scatter(acc, [idx_c], xv)
                plsc.addupdate_scatter(acc, [idx_c + REG], xv * xv)
            return 0

        lax.fori_loop(0, VPC, vreg_body, 0)
        return 0

    lax.fori_loop(0, NCH, chunk_body, 0)

    def red_body(r, _):
        base = r * REG
        for h in range(2):
            s = acc[pl.ds(base + h * 16, 16)]
            for l in range(1, 16):
                s = s + acc[pl.ds(base + l * KP2 + h * 16, 16)]
            res[pl.ds(r * KP2 + h * 16, 16)] = s
        return 0
    lax.fori_loop(0, NREG, red_body, 0)

    out_cp = pltpu.make_async_copy(
        res, out_hbm.at[pl.ds(wid * RES_N, RES_N)], sems.at[0])
    out_cp.start()
    out_cp.wait()


@functools.partial(
    pl.kernel,
    mesh=plsc.VectorSubcoreMesh(core_axis_name="c", subcore_axis_name="s"),
    out_type=jax.ShapeDtypeStruct((NW * RES_N,), jnp.float32),
    scratch_types=[
        pltpu.VMEM((2, CPW, PCH), jnp.float32),
        pltpu.VMEM((2, PCH), jnp.int32),
        pltpu.VMEM((ACC_N,), jnp.float32),
        pltpu.VMEM((RES_N,), jnp.float32),
        pltpu.SemaphoreType.DMA((2,)),
    ],
    compiler_params=pltpu.CompilerParams(needs_layout_passes=False),
)
def _sc_stats(x_hbm, lab_hbm, out_hbm, xbuf, lbuf, acc, res, sems):
    _sc_stats_body(x_hbm, lab_hbm, out_hbm, xbuf, lbuf, acc, res, sems)


def _apply_body(x_ref, lab_ref, stats_ref, cnt_ref, mt_ref, st_ref,
                out_ref, tab_ref, used_ref, sb_scr):
    j = pl.program_id(1)

    @pl.when(j == 0)
    def _():
        stats = stats_ref[0]
        sums = stats[:, :KP]
        sumsq = stats[:, KP2:KP2 + KP]
        cnt = cnt_ref[0]
        mean = sums / jnp.maximum(cnt, 1.0)
        var = (sumsq - cnt * mean * mean) / jnp.maximum(cnt - 1.0, 1.0)
        std = jnp.sqrt(jnp.maximum(var, 0.0)) + EPS
        used = cnt > float(COUNT)
        mt = mt_ref[...]
        st = st_ref[...]
        scale = jnp.where(used, st / std, 1.0)
        bias = jnp.where(used, mt - mean * scale, 0.0)
        sb_scr[0] = scale
        sb_scr[1] = bias
        tab_ref[0, 0] = jnp.where(used, mt, 0.0)
        tab_ref[0, 1] = jnp.where(used, st, 0.0)
        used_ref[0] = used.astype(jnp.int32)

    x = x_ref[0]
    lab = lab_ref[0]
    klass = jax.lax.broadcasted_iota(jnp.int32, (KP, T), 0)
    onehot = (klass == lab).astype(jnp.bfloat16)
    sb = sb_scr[...].reshape(2 * C, KP).astype(jnp.bfloat16)
    sb_px = jax.lax.dot_general(
        sb, onehot, (((1,), (0,)), ((), ())),
        preferred_element_type=jnp.float32)
    out_ref[0] = x * sb_px[:C, :] + sb_px[C:, :]


@jax.jit
def kernel(x_content, y_content, means_table, stds_table):
    xf = x_content.reshape(B * C * HW)
    labs_flat = y_content.reshape(B * HW)
    raw = _sc_stats(xf, labs_flat)

    r = raw.reshape(B, GPB, NREG, KP2)
    statsf = r[:, :, :2 * CPW, :].reshape(B, C, 2 * KP2)
    cnt = r[:, 0, NREG - 1, :KP].reshape(B, 1, KP)

    labs = y_content.reshape(B * NB, 1, T)
    mt_t = jnp.zeros((C, KP), jnp.float32).at[:, :NUM_CLASSES].set(means_table.T)
    st_t = jnp.zeros((C, KP), jnp.float32).at[:, :NUM_CLASSES].set(stds_table.T)

    out_flat, tab, used_i = pl.pallas_call(
        _apply_body,
        grid=(B, NB),
        in_specs=[
            pl.BlockSpec((1, C, T), lambda b, j: (b, 0, j)),
            pl.BlockSpec((1, 1, T), lambda b, j: (b * NB + j, 0, 0)),
            pl.BlockSpec((1, C, 2 * KP2), lambda b, j: (b, 0, 0)),
            pl.BlockSpec((1, 1, KP), lambda b, j: (b, 0, 0)),
            pl.BlockSpec((C, KP), lambda b, j: (0, 0)),
            pl.BlockSpec((C, KP), lambda b, j: (0, 0)),
        ],
        out_specs=[
            pl.BlockSpec((1, C, T), lambda b, j: (b, 0, j)),
            pl.BlockSpec((1, 2, C, KP), lambda b, j: (b, 0, 0, 0)),
            pl.BlockSpec((1, 1, KP), lambda b, j: (b, 0, 0)),
        ],
        out_shape=[
            jax.ShapeDtypeStruct((B, C, HW), jnp.float32),
            jax.ShapeDtypeStruct((B, 2, C, KP), jnp.float32),
            jax.ShapeDtypeStruct((B, 1, KP), jnp.int32),
        ],
        scratch_shapes=[pltpu.VMEM((2, C, KP), jnp.float32)],
        compiler_params=pltpu.CompilerParams(
            dimension_semantics=("arbitrary", "arbitrary")),
    )(x_content.reshape(B, C, HW), labs, statsf, cnt, mt_t, st_t)

    out = out_flat.reshape(B, C, H, W)
    sm = tab[:, 0].transpose(0, 2, 1)[:, :NUM_CLASSES, :]
    ss = tab[:, 1].transpose(0, 2, 1)[:, :NUM_CLASSES, :]
    used = used_i[:, 0, :NUM_CLASSES] != 0
    return out, sm, ss, used


# speedup vs baseline: 1.0895x; 1.0895x over previous
import functools

import jax
import jax.numpy as jnp
from jax import lax
from jax.experimental import pallas as pl
from jax.experimental.pallas import tpu as pltpu
from jax.experimental.pallas import tpu_sc as plsc

NUM_CLASSES = 19
KP = 24
EPS = 1e-05
COUNT = 6
B, C, H, W = 4, 96, 224, 224
HW = H * W
T = 7168
NB = HW // T

NW = 32
GPB = NW // B
C_SC = 24
C_TC = C - C_SC
RG = 3
CRG = C_TC // RG
CPW = C_SC // GPB
PCH = 1792
NCH = HW // PCH
VPC = PCH // 16
KP2 = 32
REG = 16 * KP2
NREG = CPW * 2 + 1
ACC_N = NREG * REG
RES_N = NREG * KP2


def _sc_stats_body(x_hbm, lab_hbm, out_hbm, xbuf, lbuf, acc, res, sems):
    wid = lax.axis_index("s") * 2 + lax.axis_index("c")
    b = wid // GPB
    g = wid % GPB
    row0 = b * C + C_TC + g * CPW
    lane32 = lax.iota(jnp.int32, 16) * KP2
    ones = jnp.ones((16,), jnp.float32)
    zeros = jnp.zeros((16,), jnp.float32)

    def zero_body(i, _):
        acc[pl.ds(i * 16, 16)] = zeros
        return 0
    lax.fori_loop(0, ACC_N // 16, zero_body, 0)

    def start_chunk(t, slot):
        for c in range(CPW):
            pltpu.make_async_copy(
                x_hbm.at[pl.ds((row0 + c) * HW + t * PCH, PCH)],
                xbuf.at[slot, c], sems.at[slot]).start()
        pltpu.make_async_copy(
            lab_hbm.at[pl.ds(b * HW + t * PCH, PCH)],
            lbuf.at[slot], sems.at[slot]).start()

    start_chunk(0, 0)

    def wait_chunk(slot):
        for c in range(CPW):
            pltpu.make_async_copy(
                x_hbm.at[pl.ds(0, PCH)],
                xbuf.at[slot, c], sems.at[slot]).wait()
        pltpu.make_async_copy(
            lab_hbm.at[pl.ds(0, PCH)],
            lbuf.at[slot], sems.at[slot]).wait()

    def chunk_body(t, _):
        slot = lax.rem(t, 2)

        @pl.when(t + 1 < NCH)
        def _():
            start_chunk(t + 1, 1 - slot)

        wait_chunk(slot)

        def vreg_body(v, _):
            pv = v * 16
            labv = lbuf[slot, pl.ds(pv, 16)]
            idx0 = lax.iota(jnp.int32, 16) * KP2 + labv
            plsc.addupdate_scatter(
                acc, [idx0 + ((NREG - 1) * REG)], jnp.ones((16,), jnp.float32))
            for c in range(CPW):
                xv = xbuf[slot, c, pl.ds(pv, 16)]
                idx_c = idx0 + (2 * c * REG)
                plsc.addupdate_scatter(acc, [idx_c], xv)
                plsc.addupdate_scatter(acc, [idx_c + REG], xv * xv)
            return 0

        lax.fori_loop(0, VPC, vreg_body, 0)
        return 0

    lax.fori_loop(0, NCH, chunk_body, 0)

    def red_body(r, _):
        base = r * REG
        for h in range(2):
            s = acc[pl.ds(base + h * 16, 16)]
            for l in range(1, 16):
                s = s + acc[pl.ds(base + l * KP2 + h * 16, 16)]
            res[pl.ds(r * KP2 + h * 16, 16)] = s
        return 0
    lax.fori_loop(0, NREG, red_body, 0)

    out_cp = pltpu.make_async_copy(
        res, out_hbm.at[pl.ds(wid * RES_N, RES_N)], sems.at[0])
    out_cp.start()
    out_cp.wait()


@functools.partial(
    pl.kernel,
    mesh=plsc.VectorSubcoreMesh(core_axis_name="c", subcore_axis_name="s"),
    out_type=jax.ShapeDtypeStruct((NW * RES_N,), jnp.float32),
    scratch_types=[
        pltpu.VMEM((2, CPW + 1, PCH), jnp.float32),
        pltpu.VMEM((2, PCH), jnp.int32),
        pltpu.VMEM((ACC_N,), jnp.float32),
        pltpu.VMEM((RES_N,), jnp.float32),
        pltpu.SemaphoreType.DMA((2,)),
    ],
    compiler_params=pltpu.CompilerParams(needs_layout_passes=False),
)
def _sc_stats(x_hbm, lab_hbm, out_hbm, xbuf, lbuf, acc, res, sems):
    _sc_stats_body(x_hbm, lab_hbm, out_hbm, xbuf, lbuf, acc, res, sems)


def _tc_stats_body(x_ref, lab_ref, raw_ref, acc_ref):
    r = pl.program_id(1)
    j = pl.program_id(2)
    x = x_ref[...]
    lab = lab_ref[0]
    klass = jax.lax.broadcasted_iota(jnp.int32, (KP, T), 0)
    onehot = (klass == lab).astype(jnp.bfloat16)
    x2 = jnp.concatenate(
        [x.astype(jnp.bfloat16), (x * x).astype(jnp.bfloat16)], axis=0)
    part = jax.lax.dot_general(
        x2, onehot, (((1,), (1,)), ((), ())),
        preferred_element_type=jnp.float32)

    @pl.when(j == 0)
    def _():
        acc_ref[...] = part

    @pl.when(j > 0)
    def _():
        acc_ref[...] += part

    @pl.when(j == NB - 1)
    def _():
        raw_ref[0, 0] = acc_ref[...]


def _apply_body(x_ref, lab_ref, stats_ref, cnt_ref, mt_ref, st_ref,
                out_ref, tab_ref, used_ref, sb_scr):
    j = pl.program_id(1)

    @pl.when(j == 0)
    def _():
        stats = stats_ref[0]
        sums = stats[:, :KP]
        sumsq = stats[:, KP2:KP2 + KP]
        cnt = cnt_ref[0]
        mean = sums / jnp.maximum(cnt, 1.0)
        var = (sumsq - cnt * mean * mean) / jnp.maximum(cnt - 1.0, 1.0)
        std = jnp.sqrt(jnp.maximum(var, 0.0)) + EPS
        used = cnt > float(COUNT)
        mt = mt_ref[...]
        st = st_ref[...]
        scale = jnp.where(used, st / std, 1.0)
        bias = jnp.where(used, mt - mean * scale, 0.0)
        sb_scr[0] = scale
        sb_scr[1] = bias
        tab_ref[0, 0] = jnp.where(used, mt, 0.0)
        tab_ref[0, 1] = jnp.where(used, st, 0.0)
        used_ref[0] = used.astype(jnp.int32)

    x = x_ref[0]
    lab = lab_ref[0]
    klass = jax.lax.broadcasted_iota(jnp.int32, (KP, T), 0)
    onehot = (klass == lab).astype(jnp.bfloat16)
    sb = sb_scr[...].reshape(2 * C, KP).astype(jnp.bfloat16)
    sb_px = jax.lax.dot_general(
        sb, onehot, (((1,), (0,)), ((), ())),
        preferred_element_type=jnp.float32)
    out_ref[0] = x * sb_px[:C, :] + sb_px[C:, :]


@jax.jit
def kernel(x_content, y_content, means_table, stds_table):
    xf = x_content.reshape(B * C * HW)
    labs_flat = y_content.reshape(B * HW)
    raw = _sc_stats(xf, labs_flat)

    labs = y_content.reshape(B * NB, 1, T)
    raw_tc = pl.pallas_call(
        _tc_stats_body,
        grid=(B, RG, NB),
        in_specs=[
            pl.BlockSpec((CRG, T), lambda b, r, j: (b * (C // CRG) + r, j)),
            pl.BlockSpec((1, 1, T), lambda b, r, j: (b * NB + j, 0, 0)),
        ],
        out_specs=pl.BlockSpec((1, 1, 2 * CRG, KP), lambda b, r, j: (b, r, 0, 0)),
        out_shape=jax.ShapeDtypeStruct((B, RG, 2 * CRG, KP), jnp.float32),
        scratch_shapes=[pltpu.VMEM((2 * CRG, KP), jnp.float32)],
        compiler_params=pltpu.CompilerParams(
            dimension_semantics=("arbitrary", "arbitrary", "arbitrary")),
    )(x_content.reshape(B * C, HW), labs)

    sums_tc = raw_tc[:, :, :CRG, :].reshape(B, C_TC, KP)
    sq_tc = raw_tc[:, :, CRG:, :].reshape(B, C_TC, KP)
    stats_tc = (jnp.zeros((B, C_TC, 2 * KP2), jnp.float32)
                .at[:, :, :KP].set(sums_tc)
                .at[:, :, KP2:KP2 + KP].set(sq_tc))
    rsc = raw.reshape(B, GPB, NREG, KP2)
    stats_sc = rsc[:, :, :2 * CPW, :].reshape(B, C_SC, 2 * KP2)
    statsf = jnp.concatenate([stats_tc, stats_sc], axis=1)
    cnt = rsc[:, 0, NREG - 1, :KP].reshape(B, 1, KP)
    mt_t = jnp.zeros((C, KP), jnp.float32).at[:, :NUM_CLASSES].set(means_table.T)
    st_t = jnp.zeros((C, KP), jnp.float32).at[:, :NUM_CLASSES].set(stds_table.T)

    out_flat, tab, used_i = pl.pallas_call(
        _apply_body,
        grid=(B, NB),
        in_specs=[
            pl.BlockSpec((1, C, T), lambda b, j: (b, 0, j)),
            pl.BlockSpec((1, 1, T), lambda b, j: (b * NB + j, 0, 0)),
            pl.BlockSpec((1, C, 2 * KP2), lambda b, j: (b, 0, 0)),
            pl.BlockSpec((1, 1, KP), lambda b, j: (b, 0, 0)),
            pl.BlockSpec((C, KP), lambda b, j: (0, 0)),
            pl.BlockSpec((C, KP), lambda b, j: (0, 0)),
        ],
        out_specs=[
            pl.BlockSpec((1, C, T), lambda b, j: (b, 0, j)),
            pl.BlockSpec((1, 2, C, KP), lambda b, j: (b, 0, 0, 0)),
            pl.BlockSpec((1, 1, KP), lambda b, j: (b, 0, 0)),
        ],
        out_shape=[
            jax.ShapeDtypeStruct((B, C, HW), jnp.float32),
            jax.ShapeDtypeStruct((B, 2, C, KP), jnp.float32),
            jax.ShapeDtypeStruct((B, 1, KP), jnp.int32),
        ],
        scratch_shapes=[pltpu.VMEM((2, C, KP), jnp.float32)],
        compiler_params=pltpu.CompilerParams(
            dimension_semantics=("arbitrary", "arbitrary")),
    )(x_content.reshape(B, C, HW), labs, statsf, cnt, mt_t, st_t)

    out = out_flat.reshape(B, C, H, W)
    sm = tab[:, 0].transpose(0, 2, 1)[:, :NUM_CLASSES, :]
    ss = tab[:, 1].transpose(0, 2, 1)[:, :NUM_CLASSES, :]
    used = used_i[:, 0, :NUM_CLASSES] != 0
    return out, sm, ss, used


# speedup vs baseline: 2.0183x; 1.8525x over previous
import functools

import jax
import jax.numpy as jnp
from jax import lax
from jax.experimental import pallas as pl
from jax.experimental.pallas import tpu as pltpu
from jax.experimental.pallas import tpu_sc as plsc

NUM_CLASSES = 19
KP = 24
EPS = 1e-05
COUNT = 6
B, C, H, W = 4, 96, 224, 224
HW = H * W
T = 7168
NB = HW // T

NW = 32
WPB = NW // B
PPW = HW // WPB
KP2 = 32
REG = 16 * KP2


def _sc_counts_body(lab_hbm, out_hbm, lbuf, acc, res, sems):
    wid = lax.axis_index("s") * 2 + lax.axis_index("c")
    b = wid // WPB
    part = wid % WPB
    zeros = jnp.zeros((16,), jnp.float32)

    def zero_body(i, _):
        acc[pl.ds(i * 16, 16)] = zeros
        return 0
    lax.fori_loop(0, REG // 16, zero_body, 0)

    cp_in = pltpu.make_async_copy(
        lab_hbm.at[pl.ds(b * HW + part * PPW, PPW)], lbuf, sems.at[0])
    cp_in.start()
    cp_in.wait()

    def vreg_body(v, _):
        labv = lbuf[pl.ds(v * 16, 16)]
        idx = lax.iota(jnp.int32, 16) * KP2 + labv
        plsc.addupdate_scatter(acc, [idx], jnp.ones((16,), jnp.float32))
        return 0
    lax.fori_loop(0, PPW // 16, vreg_body, 0)

    for h in range(2):
        s = acc[pl.ds(h * 16, 16)]
        for l in range(1, 16):
            s = s + acc[pl.ds(l * KP2 + h * 16, 16)]
        res[pl.ds(h * 16, 16)] = s

    cp_out = pltpu.make_async_copy(
        res, out_hbm.at[pl.ds(wid * KP2, KP2)], sems.at[0])
    cp_out.start()
    cp_out.wait()


@functools.partial(
    pl.kernel,
    mesh=plsc.VectorSubcoreMesh(core_axis_name="c", subcore_axis_name="s"),
    out_type=jax.ShapeDtypeStruct((NW * KP2,), jnp.float32),
    scratch_types=[
        pltpu.VMEM((PPW,), jnp.int32),
        pltpu.VMEM((REG,), jnp.float32),
        pltpu.VMEM((KP2,), jnp.float32),
        pltpu.SemaphoreType.DMA((1,)),
    ],
    compiler_params=pltpu.CompilerParams(needs_layout_passes=False),
)
def _sc_counts(lab_hbm, out_hbm, lbuf, acc, res, sems):
    _sc_counts_body(lab_hbm, out_hbm, lbuf, acc, res, sems)


def _mega_body(x_ref, lab_ref, cnt_ref, mt_ref, st_ref,
               out_ref, tab_ref, used_ref, xs, acc, sb_scr):
    p = pl.program_id(1)
    j = pl.program_id(2)
    lab = lab_ref[0]
    klass = jax.lax.broadcasted_iota(jnp.int32, (KP, T), 0)
    onehot = (klass == lab).astype(jnp.bfloat16)

    @pl.when(p == 0)
    def _():
        x = x_ref[0]
        xs[:, pl.ds(j * T, T)] = x
        x2 = jnp.concatenate(
            [x.astype(jnp.bfloat16), (x * x).astype(jnp.bfloat16)], axis=0)
        part = jax.lax.dot_general(
            x2, onehot, (((1,), (1,)), ((), ())),
            preferred_element_type=jnp.float32)

        @pl.when(j == 0)
        def _():
            acc[...] = part

        @pl.when(j > 0)
        def _():
            acc[...] += part

        @pl.when(j == NB - 1)
        def _():
            stats = acc[...]
            sums = stats[:C, :]
            sumsq = stats[C:, :]
            cnt = jnp.sum(cnt_ref[0], axis=0, keepdims=True)[:, :KP]
            mean = sums / jnp.maximum(cnt, 1.0)
            var = (sumsq - cnt * mean * mean) / jnp.maximum(cnt - 1.0, 1.0)
            std = jnp.sqrt(jnp.maximum(var, 0.0)) + EPS
            used = cnt > float(COUNT)
            mt = mt_ref[...]
            st = st_ref[...]
            scale = jnp.where(used, st / std, 1.0)
            bias = jnp.where(used, mt - mean * scale, 0.0)
            sb_scr[0] = scale
            sb_scr[1] = bias
            tab_ref[0, 0] = jnp.where(used, mt, 0.0)
            tab_ref[0, 1] = jnp.where(used, st, 0.0)
            used_ref[0] = used.astype(jnp.int32)

    @pl.when(p == 1)
    def _():
        xv = xs[:, pl.ds(j * T, T)]
        sb = sb_scr[...].reshape(2 * C, KP).astype(jnp.bfloat16)
        sb_px = jax.lax.dot_general(
            sb, onehot, (((1,), (0,)), ((), ())),
            preferred_element_type=jnp.float32)
        out_ref[0] = xv * sb_px[:C, :] + sb_px[C:, :]


@jax.jit
def kernel(x_content, y_content, means_table, stds_table):
    labs_flat = y_content.reshape(B * HW)
    cnt_raw = _sc_counts(labs_flat).reshape(B, WPB, KP2)

    labs = y_content.reshape(B * NB, 1, T)
    mt_t = jnp.zeros((C, KP), jnp.float32).at[:, :NUM_CLASSES].set(means_table.T)
    st_t = jnp.zeros((C, KP), jnp.float32).at[:, :NUM_CLASSES].set(stds_table.T)

    out_flat, tab, used_i = pl.pallas_call(
        _mega_body,
        grid=(B, 2, NB),
        in_specs=[
            pl.BlockSpec((1, C, T),
                         lambda b, p, j: (b, 0, lax.select(p == 0, j, NB - 1))),
            pl.BlockSpec((1, 1, T), lambda b, p, j: (b * NB + j, 0, 0)),
            pl.BlockSpec((1, WPB, KP2), lambda b, p, j: (b, 0, 0)),
            pl.BlockSpec((C, KP), lambda b, p, j: (0, 0)),
            pl.BlockSpec((C, KP), lambda b, p, j: (0, 0)),
        ],
        out_specs=[
            pl.BlockSpec((1, C, T), lambda b, p, j: (b, 0, j * p)),
            pl.BlockSpec((1, 2, C, KP), lambda b, p, j: (b, 0, 0, 0)),
            pl.BlockSpec((1, 1, KP), lambda b, p, j: (b, 0, 0)),
        ],
        out_shape=[
            jax.ShapeDtypeStruct((B, C, HW), jnp.float32),
            jax.ShapeDtypeStruct((B, 2, C, KP), jnp.float32),
            jax.ShapeDtypeStruct((B, 1, KP), jnp.int32),
        ],
        scratch_shapes=[
            pltpu.VMEM((C, HW), jnp.float32),
            pltpu.VMEM((2 * C, KP), jnp.float32),
            pltpu.VMEM((2, C, KP), jnp.float32),
        ],
        compiler_params=pltpu.CompilerParams(
            dimension_semantics=("arbitrary", "arbitrary", "arbitrary")),
    )(x_content.reshape(B, C, HW), labs, cnt_raw, mt_t, st_t)

    out = out_flat.reshape(B, C, H, W)
    sm = tab[:, 0].transpose(0, 2, 1)[:, :NUM_CLASSES, :]
    ss = tab[:, 1].transpose(0, 2, 1)[:, :NUM_CLASSES, :]
    used = used_i[:, 0, :NUM_CLASSES] != 0
    return out, sm, ss, used


# speedup vs baseline: 2.0907x; 1.0359x over previous
import functools

import jax
import jax.numpy as jnp
from jax import lax
from jax.experimental import pallas as pl
from jax.experimental.pallas import tpu as pltpu
from jax.experimental.pallas import tpu_sc as plsc

NUM_CLASSES = 19
KP = 24
EPS = 1e-05
COUNT = 6
B, C, H, W = 4, 96, 224, 224
HW = H * W
T = 7168
NB = HW // T

NW = 32
WPB = NW // B
PPW = HW // WPB
KP2 = 32
REG = 16 * KP2


def _sc_counts_body(lab_hbm, out_hbm, lbuf, acc, res, sems):
    wid = lax.axis_index("s") * 2 + lax.axis_index("c")
    b = wid // WPB
    part = wid % WPB
    zeros = jnp.zeros((16,), jnp.float32)

    def zero_body(i, _):
        acc[pl.ds(i * 16, 16)] = zeros
        return 0
    lax.fori_loop(0, REG // 16, zero_body, 0)

    cp_in = pltpu.make_async_copy(
        lab_hbm.at[pl.ds(b * HW + part * PPW, PPW)], lbuf, sems.at[0])
    cp_in.start()
    cp_in.wait()

    def vreg_body(v, _):
        labv = lbuf[pl.ds(v * 16, 16)]
        idx = lax.iota(jnp.int32, 16) * KP2 + labv
        plsc.addupdate_scatter(acc, [idx], jnp.ones((16,), jnp.float32))
        return 0
    lax.fori_loop(0, PPW // 16, vreg_body, 0)

    for h in range(2):
        s = acc[pl.ds(h * 16, 16)]
        for l in range(1, 16):
            s = s + acc[pl.ds(l * KP2 + h * 16, 16)]
        res[pl.ds(h * 16, 16)] = s

    cp_out = pltpu.make_async_copy(
        res, out_hbm.at[pl.ds(wid * KP2, KP2)], sems.at[0])
    cp_out.start()
    cp_out.wait()


@functools.partial(
    pl.kernel,
    mesh=plsc.VectorSubcoreMesh(core_axis_name="c", subcore_axis_name="s"),
    out_type=jax.ShapeDtypeStruct((NW * KP2,), jnp.float32),
    scratch_types=[
        pltpu.VMEM((PPW,), jnp.int32),
        pltpu.VMEM((REG,), jnp.float32),
        pltpu.VMEM((KP2,), jnp.float32),
        pltpu.SemaphoreType.DMA((1,)),
    ],
    compiler_params=pltpu.CompilerParams(needs_layout_passes=False),
)
def _sc_counts(lab_hbm, out_hbm, lbuf, acc, res, sems):
    _sc_counts_body(lab_hbm, out_hbm, lbuf, acc, res, sems)


def _mega_body(x_ref, laba_ref, labb_ref, cnt_ref, mt_ref, st_ref,
               out_ref, tab_ref, used_ref, xs, acc, sb_scr):
    s = pl.program_id(0)
    j = pl.program_id(1)
    sp = lax.rem(s, 2)
    spm = lax.rem(s + 1, 2)

    @pl.when(s < B)
    def _():
        lab = laba_ref[0]
        klass = jax.lax.broadcasted_iota(jnp.int32, (KP, T), 0)
        onehot = (klass == lab).astype(jnp.bfloat16)
        x = x_ref[0]
        xs[sp, :, pl.ds(j * T, T)] = x
        x2 = jnp.concatenate(
            [x.astype(jnp.bfloat16), (x * x).astype(jnp.bfloat16)], axis=0)
        part = jax.lax.dot_general(
            x2, onehot, (((1,), (1,)), ((), ())),
            preferred_element_type=jnp.float32)

        @pl.when(j == 0)
        def _():
            acc[...] = part

        @pl.when(j > 0)
        def _():
            acc[...] += part

        @pl.when(j == NB - 1)
        def _():
            stats = acc[...]
            sums = stats[:C, :]
            sumsq = stats[C:, :]
            cnt = jnp.sum(cnt_ref[0], axis=0, keepdims=True)[:, :KP]
            mean = sums / jnp.maximum(cnt, 1.0)
            var = (sumsq - cnt * mean * mean) / jnp.maximum(cnt - 1.0, 1.0)
            std = jnp.sqrt(jnp.maximum(var, 0.0)) + EPS
            used = cnt > float(COUNT)
            mt = mt_ref[...]
            st = st_ref[...]
            scale = jnp.where(used, st / std, 1.0)
            bias = jnp.where(used, mt - mean * scale, 0.0)
            sb_scr[sp, 0] = scale
            sb_scr[sp, 1] = bias
            tab_ref[0, 0] = jnp.where(used, mt, 0.0)
            tab_ref[0, 1] = jnp.where(used, st, 0.0)
            used_ref[0] = used.astype(jnp.int32)

    @pl.when(s >= 1)
    def _():
        lab = labb_ref[0]
        klass = jax.lax.broadcasted_iota(jnp.int32, (KP, T), 0)
        onehot = (klass == lab).astype(jnp.bfloat16)
        xv = xs[spm, :, pl.ds(j * T, T)]
        sb = sb_scr[spm].reshape(2 * C, KP).astype(jnp.bfloat16)
        sb_px = jax.lax.dot_general(
            sb, onehot, (((1,), (0,)), ((), ())),
            preferred_element_type=jnp.float32)
        out_ref[0] = xv * sb_px[:C, :] + sb_px[C:, :]


@jax.jit
def kernel(x_content, y_content, means_table, stds_table):
    labs_flat = y_content.reshape(B * HW)
    cnt_raw = _sc_counts(labs_flat).reshape(B, WPB, KP2)

    labs = y_content.reshape(B * NB, 1, T)
    mt_t = jnp.zeros((C, KP), jnp.float32).at[:, :NUM_CLASSES].set(means_table.T)
    st_t = jnp.zeros((C, KP), jnp.float32).at[:, :NUM_CLASSES].set(stds_table.T)

    out_flat, tab, used_i = pl.pallas_call(
        _mega_body,
        grid=(B + 1, NB),
        in_specs=[
            pl.BlockSpec(
                (1, C, T),
                lambda s, j: (jnp.minimum(s, B - 1), 0,
                              lax.select(s < B, j, NB - 1))),
            pl.BlockSpec(
                (1, 1, T),
                lambda s, j: (jnp.minimum(s, B - 1) * NB + j, 0, 0)),
            pl.BlockSpec(
                (1, 1, T),
                lambda s, j: (jnp.maximum(s - 1, 0) * NB + j, 0, 0)),
            pl.BlockSpec((1, WPB, KP2), lambda s, j: (jnp.minimum(s, B - 1), 0, 0)),
            pl.BlockSpec((C, KP), lambda s, j: (0, 0)),
            pl.BlockSpec((C, KP), lambda s, j: (0, 0)),
        ],
        out_specs=[
            pl.BlockSpec(
                (1, C, T),
                lambda s, j: (jnp.maximum(s - 1, 0), 0,
                              lax.select(s >= 1, j, 0))),
            pl.BlockSpec(
                (1, 2, C, KP),
                lambda s, j: (jnp.minimum(s, B - 1), 0, 0, 0)),
            pl.BlockSpec(
                (1, 1, KP), lambda s, j: (jnp.minimum(s, B - 1), 0, 0)),
        ],
        out_shape=[
            jax.ShapeDtypeStruct((B, C, HW), jnp.float32),
            jax.ShapeDtypeStruct((B, 2, C, KP), jnp.float32),
            jax.ShapeDtypeStruct((B, 1, KP), jnp.int32),
        ],
        scratch_shapes=[
            pltpu.VMEM((2, C, HW), jnp.float32),
            pltpu.VMEM((2 * C, KP), jnp.float32),
            pltpu.VMEM((2, 2, C, KP), jnp.float32),
        ],
        compiler_params=pltpu.CompilerParams(
            dimension_semantics=("arbitrary", "arbitrary")),
    )(x_content.reshape(B, C, HW), labs, labs, cnt_raw, mt_t, st_t)

    out = out_flat.reshape(B, C, H, W)
    sm = tab[:, 0].transpose(0, 2, 1)[:, :NUM_CLASSES, :]
    ss = tab[:, 1].transpose(0, 2, 1)[:, :NUM_CLASSES, :]
    used = used_i[:, 0, :NUM_CLASSES] != 0
    return out, sm, ss, used
